# baseline (speedup 1.0000x reference)
import functools

import jax
import jax.numpy as jnp
from jax import lax
from jax.experimental import pallas as pl
from jax.experimental.pallas import tpu as pltpu
from jax.experimental.pallas import tpu_sc as plsc

NUM_CORES = 2
NUM_SUBCORES = 16
LANES = 16
NUM_WORKERS = NUM_CORES * NUM_SUBCORES

BATCH = 16384
EMBED_DIM = 32
B_PER_W = BATCH // NUM_WORKERS
ROWS_PER_W = 2 * B_PER_W
GROUPS = B_PER_W // LANES

_mesh = plsc.VectorSubcoreMesh(core_axis_name="c", subcore_axis_name="s")


@functools.partial(
    pl.kernel,
    out_type=jax.ShapeDtypeStruct((BATCH,), jnp.float32),
    mesh=_mesh,
    scratch_types=[
        pltpu.VMEM((ROWS_PER_W,), jnp.int32),
        pltpu.VMEM((ROWS_PER_W, EMBED_DIM), jnp.float32),
        pltpu.VMEM((B_PER_W,), jnp.float32),
        pltpu.SemaphoreType.DMA,
    ],
    compiler_params=pltpu.CompilerParams(
        needs_layout_passes=False, use_tc_tiling_on_sc=False
    ),
)
def _road2vec_sc(xflat_hbm, table_hbm, out_hbm, idx_v, rows_v, out_v, sem):
    wid = lax.axis_index("s") * NUM_CORES + lax.axis_index("c")
    ibase = wid * ROWS_PER_W
    obase = wid * B_PER_W

    pltpu.sync_copy(xflat_hbm.at[pl.ds(ibase, ROWS_PER_W)], idx_v)
    pltpu.async_copy(table_hbm.at[idx_v], rows_v, sem).wait()

    lane = lax.iota(jnp.int32, LANES)

    def body(g, carry):
        r0 = (g * LANES + lane) * 2
        r1 = r0 + 1
        acc = jnp.zeros((LANES,), jnp.float32)
        for d in range(EMBED_DIM):
            col = jnp.full((LANES,), d, jnp.int32)
            u = plsc.load_gather(rows_v, [r0, col])
            v = plsc.load_gather(rows_v, [r1, col])
            acc = acc + u * v
        out_v[pl.ds(g * LANES, LANES)] = 1.0 / (1.0 + jnp.exp(-acc))
        return carry

    lax.fori_loop(0, GROUPS, body, 0)
    pltpu.sync_copy(out_v, out_hbm.at[pl.ds(obase, B_PER_W)])


def kernel(x, table):
    xflat = x.reshape(-1).astype(jnp.int32)
    return _road2vec_sc(xflat, table)
